# baseline (speedup 1.0000x reference)
import functools

import jax
import jax.numpy as jnp
from jax import lax
from jax.experimental import pallas as pl
from jax.experimental.pallas import tpu as pltpu
from jax.experimental.pallas import tpu_sc as plsc

_B = 16384
_D = 64
_O = 128
_G = 2 * _B
_NC = 2
_NS = 16
_NW = _NC * _NS
_BPW = _B // _NW
_RPW = 2 * _BPW
_CH = 128
_NCH = _RPW // _CH


def _sc_gather(table, idx1, idx2):
  mesh = plsc.VectorSubcoreMesh(core_axis_name="c", subcore_axis_name="s")

  @functools.partial(
      pl.kernel,
      mesh=mesh,
      out_type=jax.ShapeDtypeStruct((_G, _D), jnp.float32),
      scratch_types=[
          pltpu.VMEM((_BPW,), jnp.int32),
          pltpu.VMEM((_BPW,), jnp.int32),
          pltpu.VMEM((_NCH, _CH), jnp.int32),
          pltpu.VMEM((_RPW, _D), jnp.float32),
          pltpu.SemaphoreType.DMA,
      ],
      compiler_params=pltpu.CompilerParams(use_tc_tiling_on_sc=False,
                                           needs_layout_passes=False),
  )
  def gather_kernel(table_hbm, idx1_hbm, idx2_hbm, out_hbm,
                    i1_v, i2_v, cat_v, rows_v, sem):
    wid = lax.axis_index("s") * _NC + lax.axis_index("c")
    base = wid * _BPW
    pltpu.sync_copy(idx1_hbm.at[pl.ds(base, _BPW)], i1_v)
    pltpu.sync_copy(idx2_hbm.at[pl.ds(base, _BPW)], i2_v)
    lanes2 = 2 * lax.iota(jnp.int32, 16)

    def perm(v):
      return (((v >> 10) << 10) + ((v & 511) << 1) + ((v >> 9) & 1))

    for k in range(_BPW // 16):
      v1 = perm(i1_v[pl.ds(16 * k, 16)])
      v2 = perm(i2_v[pl.ds(16 * k, 16)])
      row = (32 * k) // _CH
      col0 = (32 * k) % _CH
      plsc.store_scatter(cat_v.at[row], [col0 + lanes2], v1)
      plsc.store_scatter(cat_v.at[row], [col0 + 1 + lanes2], v2)
    copies = []
    for j in range(_NCH):
      copies.append(pltpu.async_copy(
          table_hbm.at[cat_v.at[j]], rows_v.at[pl.ds(j * _CH, _CH)], sem))
    for c in copies:
      c.wait()
    pltpu.sync_copy(rows_v, out_hbm.at[pl.ds(wid * _RPW, _RPW)])

  return gather_kernel(table, idx1, idx2)


_TCOL = 512
_TGRID = 98
_VP = _TGRID * 2 * _TCOL


def _tc_transpose(tableT):
  def tr_kernel(ta_ref, tb_ref, o_ref):
    o_ref[:, 0:_D] = ta_ref[...].T
    o_ref[:, _D:] = tb_ref[...].T

  return pl.pallas_call(
      tr_kernel,
      grid=(_TGRID,),
      in_specs=[
          pl.BlockSpec((_D, _TCOL), lambda j: (0, 2 * j)),
          pl.BlockSpec((_D, _TCOL), lambda j: (0, 2 * j + 1)),
      ],
      out_specs=pl.BlockSpec((_TCOL, 2 * _D), lambda j: (j, 0)),
      out_shape=jax.ShapeDtypeStruct((_VP // 2, 2 * _D), jnp.float32),
  )(tableT, tableT)


_BM = 1024


def _tc_project(cat, W, b2d):
  def mm_kernel(cat_ref, w_ref, b_ref, o_ref):
    o_ref[...] = lax.dot_general(
        cat_ref[...], w_ref[...], (((1,), (1,)), ((), ())),
        preferred_element_type=jnp.float32) + b_ref[...]

  return pl.pallas_call(
      mm_kernel,
      grid=(_B // _BM,),
      in_specs=[
          pl.BlockSpec((_BM, 2 * _D), lambda i: (i, 0)),
          pl.BlockSpec((_O, 2 * _D), lambda i: (0, 0)),
          pl.BlockSpec((1, _O), lambda i: (0, 0)),
      ],
      out_specs=pl.BlockSpec((_BM, _O), lambda i: (i, 0)),
      out_shape=jax.ShapeDtypeStruct((_B, _O), jnp.float32),
  )(cat, W, b2d)


def kernel(x, emb_table, W, b):
  xT = x.T.astype(jnp.int32)
  table_lin = _tc_transpose(emb_table.T).reshape(_VP, _D)
  rows = _sc_gather(table_lin, xT[0], xT[1])
  cat = rows.reshape(_B, 2 * _D)
  return _tc_project(cat, W, b.reshape(1, _O))


# speedup vs baseline: 1.2792x; 1.2792x over previous
import functools

import jax
import jax.numpy as jnp
from jax import lax
from jax.experimental import pallas as pl
from jax.experimental.pallas import tpu as pltpu
from jax.experimental.pallas import tpu_sc as plsc

_B = 16384
_D = 64
_O = 128
_G = 2 * _B
_NC = 2
_NS = 16
_NW = _NC * _NS
_BPW = _B // _NW
_RPW = 2 * _BPW
_CH = 128
_NCH = _RPW // _CH


def _sc_gather(table, idx1, idx2):
  mesh = plsc.VectorSubcoreMesh(core_axis_name="c", subcore_axis_name="s")

  @functools.partial(
      pl.kernel,
      mesh=mesh,
      out_type=jax.ShapeDtypeStruct((_G, _D), jnp.float32),
      scratch_types=[
          pltpu.VMEM((_BPW,), jnp.int32),
          pltpu.VMEM((_BPW,), jnp.int32),
          pltpu.VMEM((_NCH, _CH), jnp.int32),
          pltpu.VMEM((_RPW, _D), jnp.float32),
          pltpu.SemaphoreType.DMA,
      ],
      compiler_params=pltpu.CompilerParams(use_tc_tiling_on_sc=False,
                                           needs_layout_passes=False),
  )
  def gather_kernel(table_hbm, idx1_hbm, idx2_hbm, out_hbm,
                    i1_v, i2_v, cat_v, rows_v, sem):
    wid = lax.axis_index("s") * _NC + lax.axis_index("c")
    base = wid * _BPW
    pltpu.sync_copy(idx1_hbm.at[pl.ds(base, _BPW)], i1_v)
    pltpu.sync_copy(idx2_hbm.at[pl.ds(base, _BPW)], i2_v)
    lanes2 = 2 * lax.iota(jnp.int32, 16)

    def perm(v):
      return (((v >> (_TSH + 1)) << (_TSH + 1)) +
              ((v & (_TCOL - 1)) << 1) + ((v >> _TSH) & 1))

    for k in range(_BPW // 16):
      v1 = perm(i1_v[pl.ds(16 * k, 16)])
      v2 = perm(i2_v[pl.ds(16 * k, 16)])
      row = (32 * k) // _CH
      col0 = (32 * k) % _CH
      plsc.store_scatter(cat_v.at[row], [col0 + lanes2], v1)
      plsc.store_scatter(cat_v.at[row], [col0 + 1 + lanes2], v2)
    copies = []
    for j in range(_NCH):
      copies.append(pltpu.async_copy(
          table_hbm.at[cat_v.at[j]], rows_v.at[pl.ds(j * _CH, _CH)], sem))
    for c in copies:
      c.wait()
    pltpu.sync_copy(rows_v, out_hbm.at[pl.ds(wid * _RPW, _RPW)])

  return gather_kernel(table, idx1, idx2)


_TCOL = 1024
_TSH = 10
_TGRID = 49
_VP = _TGRID * 2 * _TCOL


def _tc_transpose(tableT):
  def tr_kernel(ta_ref, tb_ref, o_ref):
    o_ref[:, 0:_D] = ta_ref[...].T
    o_ref[:, _D:] = tb_ref[...].T

  return pl.pallas_call(
      tr_kernel,
      grid=(_TGRID,),
      in_specs=[
          pl.BlockSpec((_D, _TCOL), lambda j: (0, 2 * j)),
          pl.BlockSpec((_D, _TCOL), lambda j: (0, 2 * j + 1)),
      ],
      out_specs=pl.BlockSpec((_TCOL, 2 * _D), lambda j: (j, 0)),
      out_shape=jax.ShapeDtypeStruct((_VP // 2, 2 * _D), jnp.float32),
  )(tableT, tableT)


_BM = 1024


def _tc_project(cat, W, b2d):
  def mm_kernel(cat_ref, w_ref, b_ref, o_ref):
    o_ref[...] = lax.dot_general(
        cat_ref[...], w_ref[...], (((1,), (1,)), ((), ())),
        preferred_element_type=jnp.float32) + b_ref[...]

  return pl.pallas_call(
      mm_kernel,
      grid=(_B // _BM,),
      in_specs=[
          pl.BlockSpec((_BM, 2 * _D), lambda i: (i, 0)),
          pl.BlockSpec((_O, 2 * _D), lambda i: (0, 0)),
          pl.BlockSpec((1, _O), lambda i: (0, 0)),
      ],
      out_specs=pl.BlockSpec((_BM, _O), lambda i: (i, 0)),
      out_shape=jax.ShapeDtypeStruct((_B, _O), jnp.float32),
  )(cat, W, b2d)


def kernel(x, emb_table, W, b):
  xT = x.T.astype(jnp.int32)
  table_lin = _tc_transpose(emb_table.T).reshape(_VP, _D)
  rows = _sc_gather(table_lin, xT[0], xT[1])
  cat = rows.reshape(_B, 2 * _D)
  return _tc_project(cat, W, b.reshape(1, _O))


# speedup vs baseline: 1.5560x; 1.2164x over previous
import functools

import jax
import jax.numpy as jnp
from jax import lax
from jax.experimental import pallas as pl
from jax.experimental.pallas import tpu as pltpu
from jax.experimental.pallas import tpu_sc as plsc

_B = 16384
_D = 64
_O = 128
_G = 2 * _B
_NC = 2
_NS = 16
_NW = _NC * _NS
_BPW = _B // _NW
_RPW = 2 * _BPW
_CH = 128
_NCH = _RPW // _CH


def _sc_gather(table, idx1, idx2):
  mesh = plsc.VectorSubcoreMesh(core_axis_name="c", subcore_axis_name="s")

  @functools.partial(
      pl.kernel,
      mesh=mesh,
      out_type=jax.ShapeDtypeStruct((_G, _D), jnp.float32),
      scratch_types=[
          pltpu.VMEM((_BPW,), jnp.int32),
          pltpu.VMEM((_BPW,), jnp.int32),
          pltpu.VMEM((_NCH, _CH), jnp.int32),
          pltpu.VMEM((_RPW, _D), jnp.float32),
          pltpu.SemaphoreType.DMA,
      ],
      compiler_params=pltpu.CompilerParams(use_tc_tiling_on_sc=False,
                                           needs_layout_passes=False),
  )
  def gather_kernel(table_hbm, idx1_hbm, idx2_hbm, out_hbm,
                    i1_v, i2_v, cat_v, rows_v, sem):
    wid = lax.axis_index("s") * _NC + lax.axis_index("c")
    base = wid * _BPW
    pltpu.sync_copy(idx1_hbm.at[pl.ds(base, _BPW)], i1_v)
    pltpu.sync_copy(idx2_hbm.at[pl.ds(base, _BPW)], i2_v)
    lanes2 = 2 * lax.iota(jnp.int32, 16)

    def perm(v):
      return (((v >> (_TSH + 1)) << (_TSH + 1)) +
              ((v & (_TCOL - 1)) << 1) + ((v >> _TSH) & 1))

    for k in range(_BPW // 16):
      v1 = perm(i1_v[pl.ds(16 * k, 16)])
      v2 = perm(i2_v[pl.ds(16 * k, 16)])
      row = (32 * k) // _CH
      col0 = (32 * k) % _CH
      plsc.store_scatter(cat_v.at[row], [col0 + lanes2], v1)
      plsc.store_scatter(cat_v.at[row], [col0 + 1 + lanes2], v2)
    copies = []
    for j in range(_NCH):
      copies.append(pltpu.async_copy(
          table_hbm.at[cat_v.at[j]], rows_v.at[pl.ds(j * _CH, _CH)], sem))
    for c in copies:
      c.wait()
    pltpu.sync_copy(rows_v, out_hbm.at[pl.ds(wid * _RPW, _RPW)])

  return gather_kernel(table, idx1, idx2)


_TCOL = 2048
_TSH = 11
_TGRID = 25
_TMAX = 100000 // _TCOL
_VP = _TGRID * 2 * _TCOL


def _tc_transpose(tableT):
  def tr_kernel(ta_ref, tb_ref, o_ref):
    o_ref[:, 0:_D] = ta_ref[...].T
    o_ref[:, _D:] = tb_ref[...].T

  return pl.pallas_call(
      tr_kernel,
      grid=(_TGRID,),
      in_specs=[
          pl.BlockSpec((_D, _TCOL), lambda j: (0, 2 * j)),
          pl.BlockSpec((_D, _TCOL), lambda j: (0, jnp.minimum(2 * j + 1, _TMAX))),
      ],
      out_specs=pl.BlockSpec((_TCOL, 2 * _D), lambda j: (j, 0)),
      out_shape=jax.ShapeDtypeStruct((_VP // 2, 2 * _D), jnp.float32),
  )(tableT, tableT)


_BM = 2048


def _tc_project(cat, W, b2d):
  def mm_kernel(cat_ref, w_ref, b_ref, o_ref):
    o_ref[...] = lax.dot_general(
        cat_ref[...], w_ref[...], (((1,), (1,)), ((), ())),
        preferred_element_type=jnp.float32) + b_ref[...]

  return pl.pallas_call(
      mm_kernel,
      grid=(_B // _BM,),
      in_specs=[
          pl.BlockSpec((_BM, 2 * _D), lambda i: (i, 0)),
          pl.BlockSpec((_O, 2 * _D), lambda i: (0, 0)),
          pl.BlockSpec((1, _O), lambda i: (0, 0)),
      ],
      out_specs=pl.BlockSpec((_BM, _O), lambda i: (i, 0)),
      out_shape=jax.ShapeDtypeStruct((_B, _O), jnp.float32),
  )(cat, W, b2d)


def kernel(x, emb_table, W, b):
  xT = x.T.astype(jnp.int32)
  table_lin = _tc_transpose(emb_table.T).reshape(_VP, _D)
  rows = _sc_gather(table_lin, xT[0], xT[1])
  cat = rows.reshape(_B, 2 * _D)
  return _tc_project(cat, W, b.reshape(1, _O))


# speedup vs baseline: 1.6897x; 1.0859x over previous
import functools

import jax
import jax.numpy as jnp
from jax import lax
from jax.experimental import pallas as pl
from jax.experimental.pallas import tpu as pltpu
from jax.experimental.pallas import tpu_sc as plsc

_B = 16384
_D = 64
_O = 128
_G = 2 * _B
_NC = 2
_NS = 16
_NW = _NC * _NS
_BPW = _B // _NW
_RPW = 2 * _BPW
_CH = 128
_NCH = _RPW // _CH


def _sc_gather(table, idx1, idx2):
  mesh = plsc.VectorSubcoreMesh(core_axis_name="c", subcore_axis_name="s")

  @functools.partial(
      pl.kernel,
      mesh=mesh,
      out_type=jax.ShapeDtypeStruct((_G, _D), jnp.float32),
      scratch_types=[
          pltpu.VMEM((_BPW,), jnp.int32),
          pltpu.VMEM((_BPW,), jnp.int32),
          pltpu.VMEM((_NCH, _CH), jnp.int32),
          pltpu.VMEM((_RPW, _D), jnp.float32),
          pltpu.SemaphoreType.DMA,
      ],
      compiler_params=pltpu.CompilerParams(use_tc_tiling_on_sc=False,
                                           needs_layout_passes=False),
  )
  def gather_kernel(table_hbm, idx1_hbm, idx2_hbm, out_hbm,
                    i1_v, i2_v, cat_v, rows_v, sem):
    wid = lax.axis_index("s") * _NC + lax.axis_index("c")
    base = wid * _BPW
    pltpu.sync_copy(idx1_hbm.at[pl.ds(base, _BPW)], i1_v)
    pltpu.sync_copy(idx2_hbm.at[pl.ds(base, _BPW)], i2_v)
    lanes2 = 2 * lax.iota(jnp.int32, 16)

    def perm(v):
      return (((v >> (_TSH + 1)) << (_TSH + 1)) +
              ((v & (_TCOL - 1)) << 1) + ((v >> _TSH) & 1))

    for k in range(_BPW // 16):
      v1 = perm(i1_v[pl.ds(16 * k, 16)])
      v2 = perm(i2_v[pl.ds(16 * k, 16)])
      row = (32 * k) // _CH
      col0 = (32 * k) % _CH
      plsc.store_scatter(cat_v.at[row], [col0 + lanes2], v1)
      plsc.store_scatter(cat_v.at[row], [col0 + 1 + lanes2], v2)
    copies = []
    for j in range(_NCH):
      copies.append(pltpu.async_copy(
          table_hbm.at[cat_v.at[j]], rows_v.at[pl.ds(j * _CH, _CH)], sem))
    for c in copies:
      c.wait()
    pltpu.sync_copy(rows_v, out_hbm.at[pl.ds(wid * _RPW, _RPW)])

  return gather_kernel(table, idx1, idx2)


_TCOL = 4096
_TSH = 12
_TGRID = 13
_TMAX = 100000 // _TCOL
_VP = _TGRID * 2 * _TCOL


def _tc_transpose(tableT):
  def tr_kernel(ta_ref, tb_ref, o_ref):
    o_ref[:, 0:_D] = ta_ref[...].T
    o_ref[:, _D:] = tb_ref[...].T

  return pl.pallas_call(
      tr_kernel,
      grid=(_TGRID,),
      in_specs=[
          pl.BlockSpec((_D, _TCOL), lambda j: (0, 2 * j)),
          pl.BlockSpec((_D, _TCOL), lambda j: (0, jnp.minimum(2 * j + 1, _TMAX))),
      ],
      out_specs=pl.BlockSpec((_TCOL, 2 * _D), lambda j: (j, 0)),
      out_shape=jax.ShapeDtypeStruct((_VP // 2, 2 * _D), jnp.float32),
  )(tableT, tableT)


_BM = 2048


def _tc_project(cat, W, b2d):
  def mm_kernel(cat_ref, w_ref, b_ref, o_ref):
    o_ref[...] = lax.dot_general(
        cat_ref[...], w_ref[...], (((1,), (1,)), ((), ())),
        preferred_element_type=jnp.float32) + b_ref[...]

  return pl.pallas_call(
      mm_kernel,
      grid=(_B // _BM,),
      in_specs=[
          pl.BlockSpec((_BM, 2 * _D), lambda i: (i, 0)),
          pl.BlockSpec((_O, 2 * _D), lambda i: (0, 0)),
          pl.BlockSpec((1, _O), lambda i: (0, 0)),
      ],
      out_specs=pl.BlockSpec((_BM, _O), lambda i: (i, 0)),
      out_shape=jax.ShapeDtypeStruct((_B, _O), jnp.float32),
  )(cat, W, b2d)


def kernel(x, emb_table, W, b):
  xT = x.T.astype(jnp.int32)
  table_lin = _tc_transpose(emb_table.T).reshape(_VP, _D)
  rows = _sc_gather(table_lin, xT[0], xT[1])
  cat = rows.reshape(_B, 2 * _D)
  return _tc_project(cat, W, b.reshape(1, _O))


# speedup vs baseline: 1.7544x; 1.0383x over previous
import functools

import jax
import jax.numpy as jnp
from jax import lax
from jax.experimental import pallas as pl
from jax.experimental.pallas import tpu as pltpu
from jax.experimental.pallas import tpu_sc as plsc

_B = 16384
_D = 64
_O = 128
_G = 2 * _B
_NC = 2
_NS = 16
_NW = _NC * _NS
_BPW = _B // _NW
_RPW = 2 * _BPW
_CH = 128
_NCH = _RPW // _CH


def _sc_gather(table, idx1, idx2):
  mesh = plsc.VectorSubcoreMesh(core_axis_name="c", subcore_axis_name="s")

  @functools.partial(
      pl.kernel,
      mesh=mesh,
      out_type=jax.ShapeDtypeStruct((_G, _D), jnp.float32),
      scratch_types=[
          pltpu.VMEM((_BPW,), jnp.int32),
          pltpu.VMEM((_BPW,), jnp.int32),
          pltpu.VMEM((_NCH, _CH), jnp.int32),
          pltpu.VMEM((_RPW, _D), jnp.float32),
          pltpu.SemaphoreType.DMA,
      ],
      compiler_params=pltpu.CompilerParams(use_tc_tiling_on_sc=False,
                                           needs_layout_passes=False),
  )
  def gather_kernel(table_hbm, idx1_hbm, idx2_hbm, out_hbm,
                    i1_v, i2_v, cat_v, rows_v, sem):
    wid = lax.axis_index("s") * _NC + lax.axis_index("c")
    base = wid * _BPW
    pltpu.sync_copy(idx1_hbm.at[pl.ds(base, _BPW)], i1_v)
    pltpu.sync_copy(idx2_hbm.at[pl.ds(base, _BPW)], i2_v)
    lanes2 = 2 * lax.iota(jnp.int32, 16)

    def perm(v):
      return (((v >> (_TSH + 1)) << (_TSH + 1)) +
              ((v & (_TCOL - 1)) << 1) + ((v >> _TSH) & 1))

    for k in range(_BPW // 16):
      v1 = perm(i1_v[pl.ds(16 * k, 16)])
      v2 = perm(i2_v[pl.ds(16 * k, 16)])
      row = (32 * k) // _CH
      col0 = (32 * k) % _CH
      plsc.store_scatter(cat_v.at[row], [col0 + lanes2], v1)
      plsc.store_scatter(cat_v.at[row], [col0 + 1 + lanes2], v2)
    copies = []
    for j in range(_NCH):
      copies.append(pltpu.async_copy(
          table_hbm.at[cat_v.at[j]], rows_v.at[pl.ds(j * _CH, _CH)], sem))
    for c in copies:
      c.wait()
    pltpu.sync_copy(rows_v, out_hbm.at[pl.ds(wid * _RPW, _RPW)])

  return gather_kernel(table, idx1, idx2)


_TCOL = 4096
_TSH = 12
_TGRID = 13
_TMAX = 100000 // _TCOL
_VP = _TGRID * 2 * _TCOL


def _tc_transpose(tableT):
  def tr_kernel(ta_ref, tb_ref, o_ref):
    o_ref[:, 0:_D] = ta_ref[...].T
    o_ref[:, _D:] = tb_ref[...].T

  return pl.pallas_call(
      tr_kernel,
      grid=(_TGRID,),
      in_specs=[
          pl.BlockSpec((_D, _TCOL), lambda j: (0, 2 * j)),
          pl.BlockSpec((_D, _TCOL), lambda j: (0, jnp.minimum(2 * j + 1, _TMAX))),
      ],
      out_specs=pl.BlockSpec((_TCOL, 2 * _D), lambda j: (j, 0)),
      out_shape=jax.ShapeDtypeStruct((_VP // 2, 2 * _D), jnp.float32),
  )(tableT, tableT)


_BM = 4096


def _tc_project(cat, W, b2d):
  def mm_kernel(cat_ref, w_ref, b_ref, o_ref):
    o_ref[...] = lax.dot_general(
        cat_ref[...], w_ref[...], (((1,), (1,)), ((), ())),
        preferred_element_type=jnp.float32) + b_ref[...]

  return pl.pallas_call(
      mm_kernel,
      grid=(_B // _BM,),
      in_specs=[
          pl.BlockSpec((_BM, 2 * _D), lambda i: (i, 0)),
          pl.BlockSpec((_O, 2 * _D), lambda i: (0, 0)),
          pl.BlockSpec((1, _O), lambda i: (0, 0)),
      ],
      out_specs=pl.BlockSpec((_BM, _O), lambda i: (i, 0)),
      out_shape=jax.ShapeDtypeStruct((_B, _O), jnp.float32),
  )(cat, W, b2d)


def kernel(x, emb_table, W, b):
  xT = x.T.astype(jnp.int32)
  table_lin = _tc_transpose(emb_table.T).reshape(_VP, _D)
  rows = _sc_gather(table_lin, xT[0], xT[1])
  cat = rows.reshape(_B, 2 * _D)
  return _tc_project(cat, W, b.reshape(1, _O))


# speedup vs baseline: 1.7944x; 1.0228x over previous
import functools

import jax
import jax.numpy as jnp
from jax import lax
from jax.experimental import pallas as pl
from jax.experimental.pallas import tpu as pltpu
from jax.experimental.pallas import tpu_sc as plsc

_B = 16384
_D = 64
_O = 128
_G = 2 * _B
_NC = 2
_NS = 16
_NW = _NC * _NS
_BPW = _B // _NW
_RPW = 2 * _BPW
_CH = 128
_NCH = _RPW // _CH


def _sc_gather(table, idx1, idx2):
  mesh = plsc.VectorSubcoreMesh(core_axis_name="c", subcore_axis_name="s")

  @functools.partial(
      pl.kernel,
      mesh=mesh,
      out_type=jax.ShapeDtypeStruct((_G, _D), jnp.float32),
      scratch_types=[
          pltpu.VMEM((_BPW,), jnp.int32),
          pltpu.VMEM((_BPW,), jnp.int32),
          pltpu.VMEM((_NCH, _CH), jnp.int32),
          pltpu.VMEM((_RPW, _D), jnp.float32),
          pltpu.SemaphoreType.DMA,
      ],
      compiler_params=pltpu.CompilerParams(use_tc_tiling_on_sc=False,
                                           needs_layout_passes=False),
  )
  def gather_kernel(table_hbm, idx1_hbm, idx2_hbm, out_hbm,
                    i1_v, i2_v, cat_v, rows_v, sem):
    wid = lax.axis_index("s") * _NC + lax.axis_index("c")
    base = wid * _BPW
    pltpu.sync_copy(idx1_hbm.at[pl.ds(base, _BPW)], i1_v)
    pltpu.sync_copy(idx2_hbm.at[pl.ds(base, _BPW)], i2_v)
    lanes2 = 2 * lax.iota(jnp.int32, 16)

    def perm(v):
      return (((v >> (_TSH + 1)) << (_TSH + 1)) +
              ((v & (_TCOL - 1)) << 1) + ((v >> _TSH) & 1))

    for k in range(_BPW // 16):
      v1 = perm(i1_v[pl.ds(16 * k, 16)])
      v2 = perm(i2_v[pl.ds(16 * k, 16)])
      row = (32 * k) // _CH
      col0 = (32 * k) % _CH
      plsc.store_scatter(cat_v.at[row], [col0 + lanes2], v1)
      plsc.store_scatter(cat_v.at[row], [col0 + 1 + lanes2], v2)
    copies = []
    for j in range(_NCH):
      copies.append(pltpu.async_copy(
          table_hbm.at[cat_v.at[j]], rows_v.at[pl.ds(j * _CH, _CH)], sem))
    for c in copies:
      c.wait()
    pltpu.sync_copy(rows_v, out_hbm.at[pl.ds(wid * _RPW, _RPW)])

  return gather_kernel(table, idx1, idx2)


_TCOL = 4096
_TSH = 12
_TGRID = 13
_TMAX = 100000 // _TCOL
_VP = _TGRID * 2 * _TCOL


def _tc_transpose(tableT):
  def tr_kernel(ta_ref, tb_ref, o_ref):
    o_ref[:, 0:_D] = ta_ref[...].T
    o_ref[:, _D:] = tb_ref[...].T

  return pl.pallas_call(
      tr_kernel,
      grid=(_TGRID,),
      in_specs=[
          pl.BlockSpec((_D, _TCOL), lambda j: (0, 2 * j)),
          pl.BlockSpec((_D, _TCOL), lambda j: (0, jnp.minimum(2 * j + 1, _TMAX))),
      ],
      out_specs=pl.BlockSpec((_TCOL, 2 * _D), lambda j: (j, 0)),
      out_shape=jax.ShapeDtypeStruct((_VP // 2, 2 * _D), jnp.float32),
  )(tableT, tableT)


_BM = 8192


def _tc_project(cat, W, b2d):
  def mm_kernel(cat_ref, w_ref, b_ref, o_ref):
    o_ref[...] = lax.dot_general(
        cat_ref[...], w_ref[...], (((1,), (1,)), ((), ())),
        preferred_element_type=jnp.float32) + b_ref[...]

  return pl.pallas_call(
      mm_kernel,
      grid=(_B // _BM,),
      in_specs=[
          pl.BlockSpec((_BM, 2 * _D), lambda i: (i, 0)),
          pl.BlockSpec((_O, 2 * _D), lambda i: (0, 0)),
          pl.BlockSpec((1, _O), lambda i: (0, 0)),
      ],
      out_specs=pl.BlockSpec((_BM, _O), lambda i: (i, 0)),
      out_shape=jax.ShapeDtypeStruct((_B, _O), jnp.float32),
  )(cat, W, b2d)


def kernel(x, emb_table, W, b):
  xT = x.T.astype(jnp.int32)
  table_lin = _tc_transpose(emb_table.T).reshape(_VP, _D)
  rows = _sc_gather(table_lin, xT[0], xT[1])
  cat = rows.reshape(_B, 2 * _D)
  return _tc_project(cat, W, b.reshape(1, _O))
